# baseline (speedup 1.0000x reference)
import functools

import jax
import jax.numpy as jnp
from jax import lax
from jax.experimental import pallas as pl
from jax.experimental.pallas import tpu as pltpu
from jax.experimental.pallas import tpu_sc as plsc

B = 16384
D = 64
H = 128
NU = 1000000
NCOMM = 100000

NC = 2
NS = 16
NW = NC * NS
BPW = B // NW


@functools.lru_cache(maxsize=None)
def _build_sc_gather():
    mesh = plsc.VectorSubcoreMesh(core_axis_name="c", subcore_axis_name="s")

    @functools.partial(
        pl.kernel,
        mesh=mesh,
        out_type=(
            jax.ShapeDtypeStruct((B, D), jnp.float32),
            jax.ShapeDtypeStruct((B, D), jnp.float32),
        ),
        scratch_types=[
            pltpu.VMEM((BPW,), jnp.int32),
            pltpu.SemaphoreType.DMA,
            pltpu.SemaphoreType.DMA,
        ],
        compiler_params=pltpu.CompilerParams(needs_layout_passes=False),
    )
    def sc_gather(uidx_hbm, cidx_hbm, utab_hbm, ctab_hbm, uout_hbm, cout_hbm,
                  idx_v, sem0, sem1):
        wid = lax.axis_index("s") * NC + lax.axis_index("c")
        base = wid * BPW

        for idx_hbm, tab_hbm, out_hbm, sem in ((uidx_hbm, utab_hbm, uout_hbm, sem0),
                                               (cidx_hbm, ctab_hbm, cout_hbm, sem1)):
            pltpu.sync_copy(idx_hbm.at[pl.ds(base, BPW)], idx_v)

            def fire(g, _, tab_hbm=tab_hbm, out_hbm=out_hbm, sem=sem):
                vec = idx_v[pl.ds(g * 16, 16)]
                tvec = lax.shift_right_logical(vec, 3)
                svec = lax.bitwise_and(vec, 7)
                for k in range(16):
                    pltpu.async_copy(
                        tab_hbm.at[tvec[k], svec[k]],
                        out_hbm.at[base + g * 16 + k], sem)
                return _

            lax.fori_loop(0, BPW // 16, fire, None)

        def drain(r, _):
            pltpu.make_async_copy(utab_hbm.at[0, 0], uout_hbm.at[base + r], sem0).wait()
            pltpu.make_async_copy(ctab_hbm.at[0, 0], cout_hbm.at[base + r], sem1).wait()
            return _

        lax.fori_loop(0, BPW, drain, None)

    return sc_gather


BM = 2048


def _mlp_body(ue_r, ce_r, ut_r, ct_r, wtag_r, btag_r, w1_r, b1_r, w2_r, b2_r,
              w3_r, b3_r, out_r):
    f32 = jnp.float32
    utf = jnp.maximum(
        jnp.dot(ut_r[...], wtag_r[...], preferred_element_type=f32) + btag_r[...], 0.0)
    ctf = jnp.maximum(
        jnp.dot(ct_r[...], wtag_r[...], preferred_element_type=f32) + btag_r[...], 0.0)
    x = jnp.concatenate([ue_r[...], ce_r[...], utf, ctf], axis=1)
    h = jnp.maximum(jnp.dot(x, w1_r[...], preferred_element_type=f32) + b1_r[...], 0.0)
    h = jnp.maximum(jnp.dot(h, w2_r[...], preferred_element_type=f32) + b2_r[...], 0.0)
    z = jnp.dot(h, w3_r[...], preferred_element_type=f32) + b3_r[...]
    out_r[...] = jax.nn.sigmoid(z)


_mlp = pl.pallas_call(
    _mlp_body,
    grid=(B // BM,),
    in_specs=[
        pl.BlockSpec((BM, D), lambda i: (i, 0)),
        pl.BlockSpec((BM, D), lambda i: (i, 0)),
        pl.BlockSpec((BM, D), lambda i: (i, 0)),
        pl.BlockSpec((BM, D), lambda i: (i, 0)),
        pl.BlockSpec((D, H), lambda i: (0, 0)),
        pl.BlockSpec((1, H), lambda i: (0, 0)),
        pl.BlockSpec((2 * D + 2 * H, 2 * H), lambda i: (0, 0)),
        pl.BlockSpec((1, 2 * H), lambda i: (0, 0)),
        pl.BlockSpec((2 * H, H), lambda i: (0, 0)),
        pl.BlockSpec((1, H), lambda i: (0, 0)),
        pl.BlockSpec((H, 1), lambda i: (0, 0)),
        pl.BlockSpec((1, 1), lambda i: (0, 0)),
    ],
    out_specs=pl.BlockSpec((BM, 1), lambda i: (i, 0)),
    out_shape=jax.ShapeDtypeStruct((B, 1), jnp.float32),
)


def kernel(user_idx, community_idx, user_tag_embedding, community_tag_embedding,
           user_table, community_table, W_tag, b_tag, W1, b1, W2, b2, W3, b3):
    uidx = user_idx.astype(jnp.int32)
    cidx = community_idx.astype(jnp.int32)
    utab3 = user_table.reshape(NU // 8, 8, D)
    ctab3 = community_table.reshape(NCOMM // 8, 8, D)
    ue, ce = _build_sc_gather()(uidx, cidx, utab3, ctab3)
    return _mlp(ue, ce, user_tag_embedding, community_tag_embedding,
                W_tag, b_tag.reshape(1, H), W1, b1.reshape(1, 2 * H),
                W2, b2.reshape(1, H), W3, b3.reshape(1, 1))


# speedup vs baseline: 2.5954x; 2.5954x over previous
import functools

import jax
import jax.numpy as jnp
from jax import lax
from jax.experimental import pallas as pl
from jax.experimental.pallas import tpu as pltpu
from jax.experimental.pallas import tpu_sc as plsc

B = 16384
D = 64
H = 128
NU = 1000000
NCOMM = 100000

NC = 2
NS = 16
NW = NC * NS
BPW = B // NW


@functools.lru_cache(maxsize=None)
def _build_sc_gather():
    mesh = plsc.VectorSubcoreMesh(core_axis_name="c", subcore_axis_name="s")

    @functools.partial(
        pl.kernel,
        mesh=mesh,
        out_type=(
            jax.ShapeDtypeStruct((B, D), jnp.float32),
            jax.ShapeDtypeStruct((B, D), jnp.float32),
        ),
        scratch_types=[
            pltpu.VMEM((BPW,), jnp.int32),
            pltpu.VMEM((BPW // 2, D), jnp.float32),
            pltpu.SemaphoreType.DMA,
        ],
    )
    def sc_gather(uidx_hbm, cidx_hbm, utab_hbm, ctab_hbm, uout_hbm, cout_hbm,
                  idx_v, rows_v, sem):
        wid = lax.axis_index("s") * NC + lax.axis_index("c")
        base = wid * BPW
        half = BPW // 2

        for idx_hbm, tab_hbm, out_hbm in ((uidx_hbm, utab_hbm, uout_hbm),
                                          (cidx_hbm, ctab_hbm, cout_hbm)):
            pltpu.sync_copy(idx_hbm.at[pl.ds(base, BPW)], idx_v)
            for h in range(2):

                def fire(g, _, tab_hbm=tab_hbm, h=h):
                    vec = idx_v[pl.ds(h * half + g * 16, 16)]
                    tvec = lax.shift_right_logical(vec, 3)
                    svec = lax.bitwise_and(vec, 7)
                    for k in range(16):
                        pltpu.async_copy(
                            tab_hbm.at[tvec[k], svec[k]],
                            rows_v.at[g * 16 + k], sem)
                    return _

                lax.fori_loop(0, half // 16, fire, None)

                def drain(r, _, tab_hbm=tab_hbm):
                    pltpu.make_async_copy(
                        tab_hbm.at[0, 0], rows_v.at[r], sem).wait()
                    return _

                lax.fori_loop(0, half, drain, None)
                pltpu.sync_copy(rows_v, out_hbm.at[pl.ds(base + h * half, half)])

    return sc_gather


BM = 2048


def _mlp_body(ue_r, ce_r, ut_r, ct_r, wtag_r, btag_r, w1_r, b1_r, w2_r, b2_r,
              w3_r, b3_r, out_r):
    f32 = jnp.float32
    utf = jnp.maximum(
        jnp.dot(ut_r[...], wtag_r[...], preferred_element_type=f32) + btag_r[...], 0.0)
    ctf = jnp.maximum(
        jnp.dot(ct_r[...], wtag_r[...], preferred_element_type=f32) + btag_r[...], 0.0)
    x = jnp.concatenate([ue_r[...], ce_r[...], utf, ctf], axis=1)
    h = jnp.maximum(jnp.dot(x, w1_r[...], preferred_element_type=f32) + b1_r[...], 0.0)
    h = jnp.maximum(jnp.dot(h, w2_r[...], preferred_element_type=f32) + b2_r[...], 0.0)
    z = jnp.dot(h, w3_r[...], preferred_element_type=f32) + b3_r[...]
    out_r[...] = jax.nn.sigmoid(z)


_mlp = pl.pallas_call(
    _mlp_body,
    grid=(B // BM,),
    in_specs=[
        pl.BlockSpec((BM, D), lambda i: (i, 0)),
        pl.BlockSpec((BM, D), lambda i: (i, 0)),
        pl.BlockSpec((BM, D), lambda i: (i, 0)),
        pl.BlockSpec((BM, D), lambda i: (i, 0)),
        pl.BlockSpec((D, H), lambda i: (0, 0)),
        pl.BlockSpec((1, H), lambda i: (0, 0)),
        pl.BlockSpec((2 * D + 2 * H, 2 * H), lambda i: (0, 0)),
        pl.BlockSpec((1, 2 * H), lambda i: (0, 0)),
        pl.BlockSpec((2 * H, H), lambda i: (0, 0)),
        pl.BlockSpec((1, H), lambda i: (0, 0)),
        pl.BlockSpec((H, 1), lambda i: (0, 0)),
        pl.BlockSpec((1, 1), lambda i: (0, 0)),
    ],
    out_specs=pl.BlockSpec((BM, 1), lambda i: (i, 0)),
    out_shape=jax.ShapeDtypeStruct((B, 1), jnp.float32),
)


def kernel(user_idx, community_idx, user_tag_embedding, community_tag_embedding,
           user_table, community_table, W_tag, b_tag, W1, b1, W2, b2, W3, b3):
    uidx = user_idx.astype(jnp.int32)
    cidx = community_idx.astype(jnp.int32)
    utab3 = user_table.reshape(NU // 8, 8, D)
    ctab3 = community_table.reshape(NCOMM // 8, 8, D)
    ue, ce = _build_sc_gather()(uidx, cidx, utab3, ctab3)
    return _mlp(ue, ce, user_tag_embedding, community_tag_embedding,
                W_tag, b_tag.reshape(1, H), W1, b1.reshape(1, 2 * H),
                W2, b2.reshape(1, H), W3, b3.reshape(1, 1))
